# baseline (speedup 1.0000x reference)
import dataclasses
import functools

import jax
import jax.numpy as jnp
from jax.experimental import pallas as pl
from jax.experimental.pallas import tpu as pltpu
from jax.experimental.pallas import tpu_sc as plsc

DIM = 512
NUM_HEADS = 8
HEAD_DIM = DIM // NUM_HEADS
N_WIN = 8
TOPK = 4
TAU = 2.0
VTH = 1.0
WIN = 128


def _qkv_kernel(x_ref, w_ref, b_ref, q_ref, k_ref, v_ref):
    xb = x_ref[0].astype(jnp.bfloat16)
    outs = (q_ref, k_ref, v_ref)
    for n in range(3):
        z = jax.lax.dot_general(
            xb, w_ref[:, n * DIM:(n + 1) * DIM],
            (((1,), (0,)), ((), ())),
            preferred_element_type=jnp.float32,
        )
        z = z + b_ref[0, n * DIM:(n + 1) * DIM][None, :]
        spk = (z * (1.0 / TAU) >= VTH)
        outs[n][0] = spk.astype(jnp.bfloat16)


def _attn_kernel(idx_ref, q_ref, k_ref, v_ref, wp_ref, bp_ref, o_ref,
                 kg_s, vg_s, at_s):
    b = pl.program_id(0) % 2
    scale = HEAD_DIM ** -0.5
    for w in range(N_WIN):
        for j in range(TOPK):
            src = idx_ref[b, w, j]
            kg_s[j * WIN:(j + 1) * WIN, :] = k_ref[0, pl.ds(src * WIN, WIN), :]
            vg_s[j * WIN:(j + 1) * WIN, :] = v_ref[0, pl.ds(src * WIN, WIN), :]
        for h in range(NUM_HEADS):
            c0 = h * HEAD_DIM
            qh = q_ref[0, w * WIN:(w + 1) * WIN, c0:c0 + HEAD_DIM]
            kh = kg_s[:, c0:c0 + HEAD_DIM]
            s = jax.lax.dot_general(
                qh, kh, (((1,), (1,)), ((), ())),
                preferred_element_type=jnp.float32,
            ) * scale
            m = jnp.max(s, axis=1, keepdims=True)
            p = jnp.exp(s - m)
            p = p / jnp.sum(p, axis=1, keepdims=True)
            at_s[:, c0:c0 + HEAD_DIM] = jax.lax.dot_general(
                p.astype(jnp.bfloat16), vg_s[:, c0:c0 + HEAD_DIM],
                (((1,), (0,)), ((), ())),
                preferred_element_type=jnp.float32,
            )
        z = jax.lax.dot_general(
            at_s[...].astype(jnp.bfloat16), wp_ref[...],
            (((1,), (0,)), ((), ())),
            preferred_element_type=jnp.float32,
        )
        z = z + bp_ref[0][None, :]
        spk = (z * (1.0 / TAU) >= VTH)
        o_ref[0, w * WIN:(w + 1) * WIN, :] = spk.astype(jnp.float32)


def _sc_compiler_params():
    cp = pltpu.CompilerParams()
    if "needs_layout_passes" in pltpu.CompilerParams.__dataclass_fields__:
        cp = dataclasses.replace(cp, needs_layout_passes=False)
    return cp


def _sc_partial_kernel(x_hbm, part_hbm, buf, acc, sem):
    b = jax.lax.axis_index("c")
    s = jax.lax.axis_index("s")
    w = jax.lax.rem(s, N_WIN)
    half = s // N_WIN
    for cc in range(DIM // 16):
        acc[pl.ds(cc * 16, 16)] = jnp.zeros((16,), jnp.float32)
    for t_i in range(2):
        t = 2 * half + t_i
        start = (t * 2 + b) * 1024 + w * WIN
        pltpu.async_copy(x_hbm.at[pl.ds(start, WIN)], buf, sem).wait()
        for cc in range(DIM // 16):
            sl = pl.ds(cc * 16, 16)
            tot = jax.lax.fori_loop(
                0, WIN,
                lambda r, a: a + buf[r, sl],
                jnp.zeros((16,), jnp.float32),
            )
            acc[sl] += tot
    pltpu.async_copy(acc, part_hbm.at[b, s], sem).wait()


def _sc_topk_kernel(part_hbm, idx_hbm, t2, outv_ref, sem):
    b = jax.lax.axis_index("c")
    s = jax.lax.axis_index("s")

    @pl.when(s < N_WIN)
    def _():
        pltpu.async_copy(part_hbm.at[b], t2, sem).wait()
        iota = jax.lax.broadcasted_iota(jnp.int32, (16,), 0)
        dots = []
        for wp in range(N_WIN):
            acc = jnp.zeros((16,), jnp.float32)
            for cc in range(DIM // 16):
                sl = pl.ds(cc * 16, 16)
                fw = t2[s, sl] + t2[s + N_WIN, sl]
                fp = t2[wp, sl] + t2[wp + N_WIN, sl]
                acc = acc + fw * fp
            dots.append(jnp.sum(acc))
        vall = jnp.zeros((16,), jnp.float32)
        for jp in range(N_WIN):
            vall = jnp.where(iota == jp, jnp.full((16,), dots[jp], jnp.float32), vall)
        outv = jnp.zeros((16,), jnp.int32)
        for j in range(N_WIN):
            dj = jnp.full((16,), dots[j], jnp.float32)
            higher = (vall > dj) & (iota < N_WIN)
            tie = (vall == dj) & (iota < j)
            rank = plsc.all_reduce_population_count(higher | tie)
            outv = outv + jnp.where(rank == iota, j, 0)
        outv_ref[...] = outv
        pltpu.async_copy(outv_ref, idx_hbm.at[b, s], sem).wait()


def _routing_indices(x):
    T, B, L, C = x.shape
    x2 = x.reshape(T * B * L, C)
    mesh = plsc.VectorSubcoreMesh(core_axis_name="c", subcore_axis_name="s")
    part = pl.kernel(
        _sc_partial_kernel,
        out_type=jax.ShapeDtypeStruct((B, 16, C), jnp.float32),
        mesh=mesh,
        scratch_types=[
            pltpu.VMEM((WIN, C), jnp.float32),
            pltpu.VMEM((C,), jnp.float32),
            pltpu.SemaphoreType.DMA,
        ],
        compiler_params=_sc_compiler_params(),
    )(x2)
    idx = pl.kernel(
        _sc_topk_kernel,
        out_type=jax.ShapeDtypeStruct((B, N_WIN, 16), jnp.int32),
        mesh=mesh,
        scratch_types=[
            pltpu.VMEM((16, C), jnp.float32),
            pltpu.VMEM((16,), jnp.int32),
            pltpu.SemaphoreType.DMA,
        ],
        compiler_params=_sc_compiler_params(),
    )(part)
    return idx


def kernel(x, W_qkv, b_qkv, W_proj, b_proj):
    T, B, L, C = x.shape
    TB = T * B
    idx = _routing_indices(x)

    xr = x.reshape(TB, L, C)
    wqkvT = W_qkv.T.astype(jnp.bfloat16)
    bqkv = b_qkv.reshape(1, 3 * C)
    wpT = W_proj.T.astype(jnp.bfloat16)
    bp = b_proj.reshape(1, C)

    qkv_shape = jax.ShapeDtypeStruct((TB, L, C), jnp.bfloat16)
    q, k, v = pl.pallas_call(
        _qkv_kernel,
        grid=(TB,),
        in_specs=[
            pl.BlockSpec((1, L, C), lambda i: (i, 0, 0)),
            pl.BlockSpec((C, 3 * C), lambda i: (0, 0)),
            pl.BlockSpec((1, 3 * C), lambda i: (0, 0)),
        ],
        out_specs=[
            pl.BlockSpec((1, L, C), lambda i: (i, 0, 0)),
            pl.BlockSpec((1, L, C), lambda i: (i, 0, 0)),
            pl.BlockSpec((1, L, C), lambda i: (i, 0, 0)),
        ],
        out_shape=[qkv_shape, qkv_shape, qkv_shape],
    )(xr, wqkvT, bqkv)

    grid_spec = pltpu.PrefetchScalarGridSpec(
        num_scalar_prefetch=1,
        grid=(TB,),
        in_specs=[
            pl.BlockSpec((1, L, C), lambda i, idx_ref: (i, 0, 0)),
            pl.BlockSpec((1, L, C), lambda i, idx_ref: (i, 0, 0)),
            pl.BlockSpec((1, L, C), lambda i, idx_ref: (i, 0, 0)),
            pl.BlockSpec((C, C), lambda i, idx_ref: (0, 0)),
            pl.BlockSpec((1, C), lambda i, idx_ref: (0, 0)),
        ],
        out_specs=pl.BlockSpec((1, L, C), lambda i, idx_ref: (i, 0, 0)),
        scratch_shapes=[
            pltpu.VMEM((TOPK * WIN, C), jnp.bfloat16),
            pltpu.VMEM((TOPK * WIN, C), jnp.bfloat16),
            pltpu.VMEM((WIN, C), jnp.float32),
        ],
    )
    out = pl.pallas_call(
        _attn_kernel,
        grid_spec=grid_spec,
        out_shape=jax.ShapeDtypeStruct((TB, L, C), jnp.float32),
    )(idx, q, k, v, wpT, bp)
    return out.reshape(T, B, L, C)


# speedup vs baseline: 1.0706x; 1.0706x over previous
import dataclasses
import functools

import jax
import jax.numpy as jnp
from jax.experimental import pallas as pl
from jax.experimental.pallas import tpu as pltpu
from jax.experimental.pallas import tpu_sc as plsc

DIM = 512
NUM_HEADS = 8
HEAD_DIM = DIM // NUM_HEADS
N_WIN = 8
TOPK = 4
TAU = 2.0
VTH = 1.0
WIN = 128


def _fused_kernel(idx_ref, x_ref, w_ref, b_ref, wp_ref, bp_ref, o_ref,
                  qkv_s, kg_s, vg_s, at_s):
    b = pl.program_id(0) % 2
    scale = HEAD_DIM ** -0.5
    xb = x_ref[0].astype(jnp.bfloat16)
    for n in range(3):
        z = jax.lax.dot_general(
            xb, w_ref[:, n * DIM:(n + 1) * DIM],
            (((1,), (0,)), ((), ())),
            preferred_element_type=jnp.float32,
        )
        z = z + b_ref[0, n * DIM:(n + 1) * DIM][None, :]
        spk = (z * (1.0 / TAU) >= VTH)
        qkv_s[:, n * DIM:(n + 1) * DIM] = spk.astype(jnp.bfloat16)
    for w in range(N_WIN):
        for j in range(TOPK):
            src = idx_ref[b, w, j]
            kg_s[j * WIN:(j + 1) * WIN, :] = qkv_s[pl.ds(src * WIN, WIN),
                                                   DIM:2 * DIM]
            vg_s[j * WIN:(j + 1) * WIN, :] = qkv_s[pl.ds(src * WIN, WIN),
                                                   2 * DIM:3 * DIM]
        for h in range(NUM_HEADS):
            c0 = h * HEAD_DIM
            qh = qkv_s[w * WIN:(w + 1) * WIN, c0:c0 + HEAD_DIM]
            kh = kg_s[:, c0:c0 + HEAD_DIM]
            s = jax.lax.dot_general(
                qh, kh, (((1,), (1,)), ((), ())),
                preferred_element_type=jnp.float32,
            ) * scale
            m = jnp.max(s, axis=1, keepdims=True)
            p = jnp.exp(s - m)
            p = p / jnp.sum(p, axis=1, keepdims=True)
            at_s[:, c0:c0 + HEAD_DIM] = jax.lax.dot_general(
                p.astype(jnp.bfloat16), vg_s[:, c0:c0 + HEAD_DIM],
                (((1,), (0,)), ((), ())),
                preferred_element_type=jnp.float32,
            )
        z = jax.lax.dot_general(
            at_s[...].astype(jnp.bfloat16), wp_ref[...],
            (((1,), (0,)), ((), ())),
            preferred_element_type=jnp.float32,
        )
        z = z + bp_ref[0][None, :]
        spk = (z * (1.0 / TAU) >= VTH)
        o_ref[0, w * WIN:(w + 1) * WIN, :] = spk.astype(jnp.float32)


def _sc_compiler_params():
    cp = pltpu.CompilerParams()
    if "needs_layout_passes" in pltpu.CompilerParams.__dataclass_fields__:
        cp = dataclasses.replace(cp, needs_layout_passes=False)
    return cp


def _sc_partial_kernel(x_hbm, part_hbm, buf, acc, sem):
    b = jax.lax.axis_index("c")
    s = jax.lax.axis_index("s")
    w = jax.lax.rem(s, N_WIN)
    half = s // N_WIN
    nch = DIM // 16

    def row_body(r, accs):
        return tuple(a + buf[r, pl.ds(cc * 16, 16)]
                     for cc, a in enumerate(accs))

    accs = tuple(jnp.zeros((16,), jnp.float32) for _ in range(nch))
    for t_i in range(2):
        t = 2 * half + t_i
        start = (t * 2 + b) * 1024 + w * WIN
        pltpu.async_copy(x_hbm.at[pl.ds(start, WIN)], buf, sem).wait()
        accs = jax.lax.fori_loop(0, WIN, row_body, accs)
    for cc in range(nch):
        acc[pl.ds(cc * 16, 16)] = accs[cc]
    pltpu.async_copy(acc, part_hbm.at[b, s], sem).wait()


def _sc_topk_kernel(part_hbm, idx_hbm, t2, outv_ref, sem):
    b = jax.lax.axis_index("c")
    s = jax.lax.axis_index("s")

    @pl.when(s < N_WIN)
    def _():
        pltpu.async_copy(part_hbm.at[b], t2, sem).wait()
        iota = jax.lax.broadcasted_iota(jnp.int32, (16,), 0)
        dots = []
        for wp in range(N_WIN):
            acc = jnp.zeros((16,), jnp.float32)
            for cc in range(DIM // 16):
                sl = pl.ds(cc * 16, 16)
                fw = t2[s, sl] + t2[s + N_WIN, sl]
                fp = t2[wp, sl] + t2[wp + N_WIN, sl]
                acc = acc + fw * fp
            dots.append(jnp.sum(acc))
        vall = jnp.zeros((16,), jnp.float32)
        for jp in range(N_WIN):
            vall = jnp.where(iota == jp, jnp.full((16,), dots[jp], jnp.float32), vall)
        outv = jnp.zeros((16,), jnp.int32)
        for j in range(N_WIN):
            dj = jnp.full((16,), dots[j], jnp.float32)
            higher = (vall > dj) & (iota < N_WIN)
            tie = (vall == dj) & (iota < j)
            rank = plsc.all_reduce_population_count(higher | tie)
            outv = outv + jnp.where(rank == iota, j, 0)
        outv_ref[...] = outv
        pltpu.async_copy(outv_ref, idx_hbm.at[b, s], sem).wait()


def _routing_indices(x):
    T, B, L, C = x.shape
    x2 = x.reshape(T * B * L, C)
    mesh = plsc.VectorSubcoreMesh(core_axis_name="c", subcore_axis_name="s")
    part = pl.kernel(
        _sc_partial_kernel,
        out_type=jax.ShapeDtypeStruct((B, 16, C), jnp.float32),
        mesh=mesh,
        scratch_types=[
            pltpu.VMEM((WIN, C), jnp.float32),
            pltpu.VMEM((C,), jnp.float32),
            pltpu.SemaphoreType.DMA,
        ],
        compiler_params=_sc_compiler_params(),
    )(x2)
    idx = pl.kernel(
        _sc_topk_kernel,
        out_type=jax.ShapeDtypeStruct((B, N_WIN, 16), jnp.int32),
        mesh=mesh,
        scratch_types=[
            pltpu.VMEM((16, C), jnp.float32),
            pltpu.VMEM((16,), jnp.int32),
            pltpu.SemaphoreType.DMA,
        ],
        compiler_params=_sc_compiler_params(),
    )(part)
    return idx


def kernel(x, W_qkv, b_qkv, W_proj, b_proj):
    T, B, L, C = x.shape
    TB = T * B
    idx = _routing_indices(x)

    xr = x.reshape(TB, L, C)
    wqkvT = W_qkv.T.astype(jnp.bfloat16)
    bqkv = b_qkv.reshape(1, 3 * C)
    wpT = W_proj.T.astype(jnp.bfloat16)
    bp = b_proj.reshape(1, C)

    grid_spec = pltpu.PrefetchScalarGridSpec(
        num_scalar_prefetch=1,
        grid=(TB,),
        in_specs=[
            pl.BlockSpec((1, L, C), lambda i, idx_ref: (i, 0, 0)),
            pl.BlockSpec((C, 3 * C), lambda i, idx_ref: (0, 0)),
            pl.BlockSpec((1, 3 * C), lambda i, idx_ref: (0, 0)),
            pl.BlockSpec((C, C), lambda i, idx_ref: (0, 0)),
            pl.BlockSpec((1, C), lambda i, idx_ref: (0, 0)),
        ],
        out_specs=pl.BlockSpec((1, L, C), lambda i, idx_ref: (i, 0, 0)),
        scratch_shapes=[
            pltpu.VMEM((L, 3 * C), jnp.bfloat16),
            pltpu.VMEM((TOPK * WIN, C), jnp.bfloat16),
            pltpu.VMEM((TOPK * WIN, C), jnp.bfloat16),
            pltpu.VMEM((WIN, C), jnp.float32),
        ],
    )
    out = pl.pallas_call(
        _fused_kernel,
        grid_spec=grid_spec,
        out_shape=jax.ShapeDtypeStruct((TB, L, C), jnp.float32),
    )(idx, xr, wqkvT, bqkv, wpT, bp)
    return out.reshape(T, B, L, C)


# speedup vs baseline: 2.2983x; 2.1467x over previous
import dataclasses
import functools

import jax
import jax.numpy as jnp
from jax.experimental import pallas as pl
from jax.experimental.pallas import tpu as pltpu
from jax.experimental.pallas import tpu_sc as plsc

DIM = 512
NUM_HEADS = 8
HEAD_DIM = DIM // NUM_HEADS
N_WIN = 8
TOPK = 4
TAU = 2.0
VTH = 1.0
WIN = 128


def _fused_kernel(idx_ref, x_ref, w_ref, b_ref, wp_ref, bp_ref, o_ref,
                  qkv_s, kg_s, vg_s, at_s, s_s, p_s):
    b = pl.program_id(0) % 2
    scale = HEAD_DIM ** -0.5
    xb = x_ref[0].astype(jnp.bfloat16)
    for n in range(3):
        z = jax.lax.dot_general(
            xb, w_ref[:, n * DIM:(n + 1) * DIM],
            (((1,), (0,)), ((), ())),
            preferred_element_type=jnp.float32,
        )
        z = z + b_ref[0, n * DIM:(n + 1) * DIM][None, :]
        spk = (z * (1.0 / TAU) >= VTH)
        qkv_s[:, n * DIM:(n + 1) * DIM] = spk.astype(jnp.bfloat16)
    for w in range(N_WIN):
        for j in range(TOPK):
            src = idx_ref[b, w, j]
            kg_s[j * WIN:(j + 1) * WIN, :] = qkv_s[pl.ds(src * WIN, WIN),
                                                   DIM:2 * DIM]
            vg_s[j * WIN:(j + 1) * WIN, :] = qkv_s[pl.ds(src * WIN, WIN),
                                                   2 * DIM:3 * DIM]
        for h in range(NUM_HEADS):
            c0 = h * HEAD_DIM
            qh = qkv_s[w * WIN:(w + 1) * WIN, c0:c0 + HEAD_DIM]
            kh = kg_s[:, c0:c0 + HEAD_DIM]
            s_s[h * WIN:(h + 1) * WIN, :] = jax.lax.dot_general(
                qh, kh, (((1,), (1,)), ((), ())),
                preferred_element_type=jnp.float32,
            )
        e = jnp.exp(s_s[...] * scale)
        rd = 1.0 / jnp.sum(e, axis=1, keepdims=True)
        p_s[...] = e.astype(jnp.bfloat16)
        for h in range(NUM_HEADS):
            c0 = h * HEAD_DIM
            oh = jax.lax.dot_general(
                p_s[h * WIN:(h + 1) * WIN, :], vg_s[:, c0:c0 + HEAD_DIM],
                (((1,), (0,)), ((), ())),
                preferred_element_type=jnp.float32,
            )
            at_s[:, c0:c0 + HEAD_DIM] = oh * rd[h * WIN:(h + 1) * WIN, :]
        z = jax.lax.dot_general(
            at_s[...].astype(jnp.bfloat16), wp_ref[...],
            (((1,), (0,)), ((), ())),
            preferred_element_type=jnp.float32,
        )
        z = z + bp_ref[0][None, :]
        spk = (z * (1.0 / TAU) >= VTH)
        o_ref[0, w * WIN:(w + 1) * WIN, :] = spk.astype(jnp.float32)


def _sc_compiler_params():
    cp = pltpu.CompilerParams()
    if "needs_layout_passes" in pltpu.CompilerParams.__dataclass_fields__:
        cp = dataclasses.replace(cp, needs_layout_passes=False)
    return cp


def _sc_partial_kernel(x_hbm, part_hbm, buf, acc, sem):
    b = jax.lax.axis_index("c")
    s = jax.lax.axis_index("s")
    w = jax.lax.rem(s, N_WIN)
    half = s // N_WIN
    nch = DIM // 16

    def row_body(r, accs):
        return tuple(a + buf[r, pl.ds(cc * 16, 16)]
                     for cc, a in enumerate(accs))

    accs = tuple(jnp.zeros((16,), jnp.float32) for _ in range(nch))
    for t_i in range(2):
        t = 2 * half + t_i
        start = (t * 2 + b) * 1024 + w * WIN
        pltpu.async_copy(x_hbm.at[pl.ds(start, WIN)], buf, sem).wait()
        accs = jax.lax.fori_loop(0, WIN, row_body, accs)
    for cc in range(nch):
        acc[pl.ds(cc * 16, 16)] = accs[cc]
    pltpu.async_copy(acc, part_hbm.at[b, s], sem).wait()


def _sc_topk_kernel(part_hbm, idx_hbm, t2, outv_ref, sem):
    b = jax.lax.axis_index("c")
    s = jax.lax.axis_index("s")

    @pl.when(s < N_WIN)
    def _():
        pltpu.async_copy(part_hbm.at[b], t2, sem).wait()
        iota = jax.lax.broadcasted_iota(jnp.int32, (16,), 0)
        dots = []
        for wp in range(N_WIN):
            acc = jnp.zeros((16,), jnp.float32)
            for cc in range(DIM // 16):
                sl = pl.ds(cc * 16, 16)
                fw = t2[s, sl] + t2[s + N_WIN, sl]
                fp = t2[wp, sl] + t2[wp + N_WIN, sl]
                acc = acc + fw * fp
            dots.append(jnp.sum(acc))
        vall = jnp.zeros((16,), jnp.float32)
        for jp in range(N_WIN):
            vall = jnp.where(iota == jp, jnp.full((16,), dots[jp], jnp.float32), vall)
        outv = jnp.zeros((16,), jnp.int32)
        for j in range(N_WIN):
            dj = jnp.full((16,), dots[j], jnp.float32)
            higher = (vall > dj) & (iota < N_WIN)
            tie = (vall == dj) & (iota < j)
            rank = plsc.all_reduce_population_count(higher | tie)
            outv = outv + jnp.where(rank == iota, j, 0)
        outv_ref[...] = outv
        pltpu.async_copy(outv_ref, idx_hbm.at[b, s], sem).wait()


def _routing_indices(x):
    T, B, L, C = x.shape
    x2 = x.reshape(T * B * L, C)
    mesh = plsc.VectorSubcoreMesh(core_axis_name="c", subcore_axis_name="s")
    part = pl.kernel(
        _sc_partial_kernel,
        out_type=jax.ShapeDtypeStruct((B, 16, C), jnp.float32),
        mesh=mesh,
        scratch_types=[
            pltpu.VMEM((WIN, C), jnp.float32),
            pltpu.VMEM((C,), jnp.float32),
            pltpu.SemaphoreType.DMA,
        ],
        compiler_params=_sc_compiler_params(),
    )(x2)
    idx = pl.kernel(
        _sc_topk_kernel,
        out_type=jax.ShapeDtypeStruct((B, N_WIN, 16), jnp.int32),
        mesh=mesh,
        scratch_types=[
            pltpu.VMEM((16, C), jnp.float32),
            pltpu.VMEM((16,), jnp.int32),
            pltpu.SemaphoreType.DMA,
        ],
        compiler_params=_sc_compiler_params(),
    )(part)
    return idx


def kernel(x, W_qkv, b_qkv, W_proj, b_proj):
    T, B, L, C = x.shape
    TB = T * B
    idx = _routing_indices(x)

    xr = x.reshape(TB, L, C)
    wqkvT = W_qkv.T.astype(jnp.bfloat16)
    bqkv = b_qkv.reshape(1, 3 * C)
    wpT = W_proj.T.astype(jnp.bfloat16)
    bp = b_proj.reshape(1, C)

    grid_spec = pltpu.PrefetchScalarGridSpec(
        num_scalar_prefetch=1,
        grid=(TB,),
        in_specs=[
            pl.BlockSpec((1, L, C), lambda i, idx_ref: (i, 0, 0)),
            pl.BlockSpec((C, 3 * C), lambda i, idx_ref: (0, 0)),
            pl.BlockSpec((1, 3 * C), lambda i, idx_ref: (0, 0)),
            pl.BlockSpec((C, C), lambda i, idx_ref: (0, 0)),
            pl.BlockSpec((1, C), lambda i, idx_ref: (0, 0)),
        ],
        out_specs=pl.BlockSpec((1, L, C), lambda i, idx_ref: (i, 0, 0)),
        scratch_shapes=[
            pltpu.VMEM((L, 3 * C), jnp.bfloat16),
            pltpu.VMEM((TOPK * WIN, C), jnp.bfloat16),
            pltpu.VMEM((TOPK * WIN, C), jnp.bfloat16),
            pltpu.VMEM((WIN, C), jnp.float32),
            pltpu.VMEM((NUM_HEADS * WIN, TOPK * WIN), jnp.float32),
            pltpu.VMEM((NUM_HEADS * WIN, TOPK * WIN), jnp.bfloat16),
        ],
    )
    out = pl.pallas_call(
        _fused_kernel,
        grid_spec=grid_spec,
        out_shape=jax.ShapeDtypeStruct((TB, L, C), jnp.float32),
    )(idx, xr, wqkvT, bqkv, wpT, bp)
    return out.reshape(T, B, L, C)


# speedup vs baseline: 2.4959x; 1.0860x over previous
import dataclasses
import functools

import jax
import jax.numpy as jnp
from jax.experimental import pallas as pl
from jax.experimental.pallas import tpu as pltpu
from jax.experimental.pallas import tpu_sc as plsc

DIM = 512
NUM_HEADS = 8
HEAD_DIM = DIM // NUM_HEADS
N_WIN = 8
TOPK = 4
TAU = 2.0
VTH = 1.0
WIN = 128


def _fused_kernel(idx_ref, x_ref, w_ref, b_ref, wp_ref, bp_ref, o_ref,
                  qkv_s, kg_s, vg_s, at_s, s_s, p_s):
    b = pl.program_id(0) % 2
    scale = HEAD_DIM ** -0.5
    xb = x_ref[0].astype(jnp.bfloat16)
    for n in range(3):
        z = jax.lax.dot_general(
            xb, w_ref[:, n * DIM:(n + 1) * DIM],
            (((1,), (0,)), ((), ())),
            preferred_element_type=jnp.float32,
        )
        z = z + b_ref[0, n * DIM:(n + 1) * DIM][None, :]
        amp = scale if n == 0 else 1.0
        spk = jnp.where(z * (1.0 / TAU) >= VTH, amp, 0.0)
        qkv_s[:, n * DIM:(n + 1) * DIM] = spk.astype(jnp.bfloat16)
    for w in range(N_WIN):
        for j in range(TOPK):
            src = idx_ref[b, w, j]
            kg_s[j * WIN:(j + 1) * WIN, :] = qkv_s[pl.ds(src * WIN, WIN),
                                                   DIM:2 * DIM]
            vg_s[j * WIN:(j + 1) * WIN, :] = qkv_s[pl.ds(src * WIN, WIN),
                                                   2 * DIM:3 * DIM]
        for h in range(NUM_HEADS):
            c0 = h * HEAD_DIM
            qh = qkv_s[w * WIN:(w + 1) * WIN, c0:c0 + HEAD_DIM]
            kh = kg_s[:, c0:c0 + HEAD_DIM]
            s_s[h * WIN:(h + 1) * WIN, :] = jax.lax.dot_general(
                qh, kh, (((1,), (1,)), ((), ())),
                preferred_element_type=jnp.float32,
            )
        e = jnp.exp(s_s[...])
        rd = 1.0 / jnp.sum(e, axis=1, keepdims=True)
        p_s[...] = e.astype(jnp.bfloat16)
        for h in range(NUM_HEADS):
            c0 = h * HEAD_DIM
            oh = jax.lax.dot_general(
                p_s[h * WIN:(h + 1) * WIN, :], vg_s[:, c0:c0 + HEAD_DIM],
                (((1,), (0,)), ((), ())),
                preferred_element_type=jnp.float32,
            )
            at_s[:, c0:c0 + HEAD_DIM] = (
                oh * rd[h * WIN:(h + 1) * WIN, :]).astype(jnp.bfloat16)
        z = jax.lax.dot_general(
            at_s[...], wp_ref[...],
            (((1,), (0,)), ((), ())),
            preferred_element_type=jnp.float32,
        )
        z = z + bp_ref[0][None, :]
        spk = (z * (1.0 / TAU) >= VTH)
        o_ref[0, w * WIN:(w + 1) * WIN, :] = spk.astype(jnp.float32)


def _sc_compiler_params():
    cp = pltpu.CompilerParams()
    if "needs_layout_passes" in pltpu.CompilerParams.__dataclass_fields__:
        cp = dataclasses.replace(cp, needs_layout_passes=False)
    return cp


def _sc_routing_kernel(x_hbm, idx_hbm, buf0, buf1, acc, t2, outv_ref,
                       shared, sem0, sem1):
    b = jax.lax.axis_index("c")
    s = jax.lax.axis_index("s")
    w = jax.lax.rem(s, N_WIN)
    half = s // N_WIN
    nch = DIM // 16
    nrow = WIN // 2
    bufs = (buf0, buf1)
    sems = (sem0, sem1)

    def chunk_start(ci):
        t = 2 * half + ci // 2
        return (t * 2 + b) * 1024 + w * WIN + (ci % 2) * nrow

    def row_body_for(buf):
        def row_body(r, accs):
            return tuple(a + buf[r, pl.ds(cc * 16, 16)]
                         for cc, a in enumerate(accs))
        return row_body

    copies = [pltpu.async_copy(x_hbm.at[pl.ds(chunk_start(0), nrow)],
                               bufs[0], sems[0])]
    accs = tuple(jnp.zeros((16,), jnp.float32) for _ in range(nch))
    for ci in range(4):
        if ci + 1 < 4:
            copies.append(
                pltpu.async_copy(x_hbm.at[pl.ds(chunk_start(ci + 1), nrow)],
                                 bufs[(ci + 1) % 2], sems[(ci + 1) % 2]))
        copies[ci].wait()
        accs = jax.lax.fori_loop(0, nrow, row_body_for(bufs[ci % 2]), accs)
    for cc in range(nch):
        acc[pl.ds(cc * 16, 16)] = accs[cc]
    pltpu.sync_copy(acc, shared.at[s])
    plsc.subcore_barrier()

    @pl.when(s < N_WIN)
    def _():
        pltpu.sync_copy(shared, t2)
        iota = jax.lax.broadcasted_iota(jnp.int32, (16,), 0)
        dots = []
        for wp in range(N_WIN):
            dacc = jnp.zeros((16,), jnp.float32)
            for cc in range(nch):
                sl = pl.ds(cc * 16, 16)
                fw = t2[s, sl] + t2[s + N_WIN, sl]
                fp = t2[wp, sl] + t2[wp + N_WIN, sl]
                dacc = dacc + fw * fp
            dots.append(jnp.sum(dacc))
        vall = jnp.zeros((16,), jnp.float32)
        for jp in range(N_WIN):
            vall = jnp.where(iota == jp,
                             jnp.full((16,), dots[jp], jnp.float32), vall)
        outv = jnp.zeros((16,), jnp.int32)
        for j in range(N_WIN):
            dj = jnp.full((16,), dots[j], jnp.float32)
            higher = (vall > dj) & (iota < N_WIN)
            tie = (vall == dj) & (iota < j)
            rank = plsc.all_reduce_population_count(higher | tie)
            outv = outv + jnp.where(rank == iota, j, 0)
        outv_ref[...] = outv
        pltpu.async_copy(outv_ref, idx_hbm.at[b, s], sem0).wait()


def _routing_indices(x):
    T, B, L, C = x.shape
    x2 = x.reshape(T * B * L, C)
    mesh = plsc.VectorSubcoreMesh(core_axis_name="c", subcore_axis_name="s")
    idx = pl.kernel(
        _sc_routing_kernel,
        out_type=jax.ShapeDtypeStruct((B, N_WIN, 16), jnp.int32),
        mesh=mesh,
        scratch_types=[
            pltpu.VMEM((WIN // 2, C), jnp.float32),
            pltpu.VMEM((WIN // 2, C), jnp.float32),
            pltpu.VMEM((C,), jnp.float32),
            pltpu.VMEM((16, C), jnp.float32),
            pltpu.VMEM((16,), jnp.int32),
            pltpu.VMEM_SHARED((16, C), jnp.float32),
            pltpu.SemaphoreType.DMA,
            pltpu.SemaphoreType.DMA,
        ],
        compiler_params=_sc_compiler_params(),
    )(x2)
    return idx


def kernel(x, W_qkv, b_qkv, W_proj, b_proj):
    T, B, L, C = x.shape
    TB = T * B
    idx = _routing_indices(x)

    xr = x.reshape(TB, L, C)
    wqkvT = W_qkv.T.astype(jnp.bfloat16)
    bqkv = b_qkv.reshape(1, 3 * C)
    wpT = W_proj.T.astype(jnp.bfloat16)
    bp = b_proj.reshape(1, C)

    grid_spec = pltpu.PrefetchScalarGridSpec(
        num_scalar_prefetch=1,
        grid=(TB,),
        in_specs=[
            pl.BlockSpec((1, L, C), lambda i, idx_ref: (i, 0, 0)),
            pl.BlockSpec((C, 3 * C), lambda i, idx_ref: (0, 0)),
            pl.BlockSpec((1, 3 * C), lambda i, idx_ref: (0, 0)),
            pl.BlockSpec((C, C), lambda i, idx_ref: (0, 0)),
            pl.BlockSpec((1, C), lambda i, idx_ref: (0, 0)),
        ],
        out_specs=pl.BlockSpec((1, L, C), lambda i, idx_ref: (i, 0, 0)),
        scratch_shapes=[
            pltpu.VMEM((L, 3 * C), jnp.bfloat16),
            pltpu.VMEM((TOPK * WIN, C), jnp.bfloat16),
            pltpu.VMEM((TOPK * WIN, C), jnp.bfloat16),
            pltpu.VMEM((WIN, C), jnp.bfloat16),
            pltpu.VMEM((NUM_HEADS * WIN, TOPK * WIN), jnp.float32),
            pltpu.VMEM((NUM_HEADS * WIN, TOPK * WIN), jnp.bfloat16),
        ],
    )
    out = pl.pallas_call(
        _fused_kernel,
        grid_spec=grid_spec,
        out_shape=jax.ShapeDtypeStruct((TB, L, C), jnp.float32),
    )(idx, xr, wqkvT, bqkv, wpT, bp)
    return out.reshape(T, B, L, C)


# speedup vs baseline: 2.7991x; 1.1215x over previous
import dataclasses
import functools

import jax
import jax.numpy as jnp
from jax.experimental import pallas as pl
from jax.experimental.pallas import tpu as pltpu
from jax.experimental.pallas import tpu_sc as plsc

DIM = 512
NUM_HEADS = 8
HEAD_DIM = DIM // NUM_HEADS
N_WIN = 8
TOPK = 4
TAU = 2.0
VTH = 1.0
WIN = 128


def _fused_kernel(idx_ref, x_ref, w_ref, b_ref, wp_ref, bp_ref, o_ref,
                  qkv_s, kg_s, vg_s, at_s, s_s, p_s):
    b = pl.program_id(0) % 2
    scale = HEAD_DIM ** -0.5
    xb = x_ref[0].astype(jnp.bfloat16)
    for n in range(3):
        z = jax.lax.dot_general(
            xb, w_ref[:, n * DIM:(n + 1) * DIM],
            (((1,), (0,)), ((), ())),
            preferred_element_type=jnp.float32,
        )
        z = z + b_ref[0, n * DIM:(n + 1) * DIM][None, :]
        amp = scale if n == 0 else 1.0
        spk = jnp.where(z * (1.0 / TAU) >= VTH, amp, 0.0)
        qkv_s[:, n * DIM:(n + 1) * DIM] = spk.astype(jnp.float8_e5m2)
    for w in range(N_WIN):
        for j in range(TOPK):
            src = idx_ref[b, w, j]
            kg_s[j * WIN:(j + 1) * WIN, :] = qkv_s[pl.ds(src * WIN, WIN),
                                                   DIM:2 * DIM]
            vg_s[j * WIN:(j + 1) * WIN, :] = qkv_s[pl.ds(src * WIN, WIN),
                                                   2 * DIM:3 * DIM]
        for h in range(NUM_HEADS):
            c0 = h * HEAD_DIM
            qh = qkv_s[w * WIN:(w + 1) * WIN, c0:c0 + HEAD_DIM]
            kh = kg_s[:, c0:c0 + HEAD_DIM]
            s_s[h * WIN:(h + 1) * WIN, :] = jax.lax.dot_general(
                qh, kh, (((1,), (1,)), ((), ())),
                preferred_element_type=jnp.float32,
            )
        e = jnp.exp(s_s[...])
        rd = 1.0 / jnp.sum(e, axis=1, keepdims=True)
        p_s[...] = e.astype(jnp.float8_e5m2)
        for h in range(NUM_HEADS):
            c0 = h * HEAD_DIM
            oh = jax.lax.dot_general(
                p_s[h * WIN:(h + 1) * WIN, :], vg_s[:, c0:c0 + HEAD_DIM],
                (((1,), (0,)), ((), ())),
                preferred_element_type=jnp.float32,
            )
            at_s[:, c0:c0 + HEAD_DIM] = (
                oh * rd[h * WIN:(h + 1) * WIN, :]).astype(jnp.bfloat16)
        z = jax.lax.dot_general(
            at_s[...], wp_ref[...],
            (((1,), (0,)), ((), ())),
            preferred_element_type=jnp.float32,
        )
        z = z + bp_ref[0][None, :]
        spk = (z * (1.0 / TAU) >= VTH)
        o_ref[0, w * WIN:(w + 1) * WIN, :] = spk.astype(jnp.float32)


def _sc_compiler_params():
    cp = pltpu.CompilerParams()
    if "needs_layout_passes" in pltpu.CompilerParams.__dataclass_fields__:
        cp = dataclasses.replace(cp, needs_layout_passes=False)
    return cp


def _sc_routing_kernel(x_hbm, idx_hbm, buf0, buf1, acc, t2, outv_ref,
                       shared, sem0, sem1):
    b = jax.lax.axis_index("c")
    s = jax.lax.axis_index("s")
    w = jax.lax.rem(s, N_WIN)
    half = s // N_WIN
    nch = DIM // 16
    nrow = WIN // 2
    bufs = (buf0, buf1)
    sems = (sem0, sem1)

    def chunk_start(ci):
        t = 2 * half + ci // 2
        return (t * 2 + b) * 1024 + w * WIN + (ci % 2) * nrow

    def row_body_for(buf):
        def row_body(r, accs):
            return tuple(a + buf[r, pl.ds(cc * 16, 16)]
                         for cc, a in enumerate(accs))
        return row_body

    copies = [pltpu.async_copy(x_hbm.at[pl.ds(chunk_start(0), nrow)],
                               bufs[0], sems[0])]
    accs = tuple(jnp.zeros((16,), jnp.float32) for _ in range(nch))
    for ci in range(4):
        if ci + 1 < 4:
            copies.append(
                pltpu.async_copy(x_hbm.at[pl.ds(chunk_start(ci + 1), nrow)],
                                 bufs[(ci + 1) % 2], sems[(ci + 1) % 2]))
        copies[ci].wait()
        accs = jax.lax.fori_loop(0, nrow, row_body_for(bufs[ci % 2]), accs)
    for cc in range(nch):
        acc[pl.ds(cc * 16, 16)] = accs[cc]
    pltpu.sync_copy(acc, shared.at[s])
    plsc.subcore_barrier()

    @pl.when(s < N_WIN)
    def _():
        pltpu.sync_copy(shared, t2)
        iota = jax.lax.broadcasted_iota(jnp.int32, (16,), 0)
        dots = []
        for wp in range(N_WIN):
            dacc = jnp.zeros((16,), jnp.float32)
            for cc in range(nch):
                sl = pl.ds(cc * 16, 16)
                fw = t2[s, sl] + t2[s + N_WIN, sl]
                fp = t2[wp, sl] + t2[wp + N_WIN, sl]
                dacc = dacc + fw * fp
            dots.append(jnp.sum(dacc))
        vall = jnp.zeros((16,), jnp.float32)
        for jp in range(N_WIN):
            vall = jnp.where(iota == jp,
                             jnp.full((16,), dots[jp], jnp.float32), vall)
        outv = jnp.zeros((16,), jnp.int32)
        for j in range(N_WIN):
            dj = jnp.full((16,), dots[j], jnp.float32)
            higher = (vall > dj) & (iota < N_WIN)
            tie = (vall == dj) & (iota < j)
            rank = plsc.all_reduce_population_count(higher | tie)
            outv = outv + jnp.where(rank == iota, j, 0)
        outv_ref[...] = outv
        pltpu.async_copy(outv_ref, idx_hbm.at[b, s], sem0).wait()


def _routing_indices(x):
    T, B, L, C = x.shape
    x2 = x.reshape(T * B * L, C)
    mesh = plsc.VectorSubcoreMesh(core_axis_name="c", subcore_axis_name="s")
    idx = pl.kernel(
        _sc_routing_kernel,
        out_type=jax.ShapeDtypeStruct((B, N_WIN, 16), jnp.int32),
        mesh=mesh,
        scratch_types=[
            pltpu.VMEM((WIN // 2, C), jnp.float32),
            pltpu.VMEM((WIN // 2, C), jnp.float32),
            pltpu.VMEM((C,), jnp.float32),
            pltpu.VMEM((16, C), jnp.float32),
            pltpu.VMEM((16,), jnp.int32),
            pltpu.VMEM_SHARED((16, C), jnp.float32),
            pltpu.SemaphoreType.DMA,
            pltpu.SemaphoreType.DMA,
        ],
        compiler_params=_sc_compiler_params(),
    )(x2)
    return idx


def kernel(x, W_qkv, b_qkv, W_proj, b_proj):
    T, B, L, C = x.shape
    TB = T * B
    idx = _routing_indices(x)

    xr = x.reshape(TB, L, C)
    wqkvT = W_qkv.T.astype(jnp.bfloat16)
    bqkv = b_qkv.reshape(1, 3 * C)
    wpT = W_proj.T.astype(jnp.bfloat16)
    bp = b_proj.reshape(1, C)

    grid_spec = pltpu.PrefetchScalarGridSpec(
        num_scalar_prefetch=1,
        grid=(TB,),
        in_specs=[
            pl.BlockSpec((1, L, C), lambda i, idx_ref: (i, 0, 0)),
            pl.BlockSpec((C, 3 * C), lambda i, idx_ref: (0, 0)),
            pl.BlockSpec((1, 3 * C), lambda i, idx_ref: (0, 0)),
            pl.BlockSpec((C, C), lambda i, idx_ref: (0, 0)),
            pl.BlockSpec((1, C), lambda i, idx_ref: (0, 0)),
        ],
        out_specs=pl.BlockSpec((1, L, C), lambda i, idx_ref: (i, 0, 0)),
        scratch_shapes=[
            pltpu.VMEM((L, 3 * C), jnp.float8_e5m2),
            pltpu.VMEM((TOPK * WIN, C), jnp.float8_e5m2),
            pltpu.VMEM((TOPK * WIN, C), jnp.float8_e5m2),
            pltpu.VMEM((WIN, C), jnp.bfloat16),
            pltpu.VMEM((NUM_HEADS * WIN, TOPK * WIN), jnp.float32),
            pltpu.VMEM((NUM_HEADS * WIN, TOPK * WIN), jnp.float8_e5m2),
        ],
    )
    out = pl.pallas_call(
        _fused_kernel,
        grid_spec=grid_spec,
        out_shape=jax.ShapeDtypeStruct((TB, L, C), jnp.float32),
    )(idx, xr, wqkvT, bqkv, wpT, bp)
    return out.reshape(T, B, L, C)
